# baseline (speedup 1.0000x reference)
import jax
import jax.numpy as jnp
import numpy as np
from jax.experimental import pallas as pl
from jax.experimental.pallas import tpu as pltpu

_RATIOS = [[4, 4], [3, 5], [5, 3], [6, 6], [5, 7], [7, 5], [8, 8], [6, 10],
           [10, 6], [7, 9], [9, 7], [7, 10], [10, 7]]
_STRIDE = 16
_FM = 28
_CAT_NUMS = [2, 3, 2]
_GROUP_RATIOS = [[0, 1, 2], [3, 4, 5], [6, 7, 8, 9, 10, 11, 12]]
_IOU_THRESH = 0.25
_CB = 6
_CBS = 128
_BIG_I = np.int32(1 << 30)

_IOTA_T = np.full((13, _FM, _FM), _BIG_I, np.int32)
_X0_T = np.full((13, _FM, _FM), 1e9, np.float32)
_Y0_T = np.full((13, _FM, _FM), 1e9, np.float32)
_X1_T = np.full((13, _FM, _FM), -1e9, np.float32)
_Y1_T = np.full((13, _FM, _FM), -1e9, np.float32)
_AR_T = np.full((13, _FM, _FM), 1.0, np.float32)
_OFFS = []
_off = 0
for _r, (_kh, _kw) in enumerate(_RATIOS):
    _H2, _W2 = _FM + 1 - _kh, _FM + 1 - _kw
    _OFFS.append(_off)
    _ii, _jj = np.meshgrid(np.arange(_H2), np.arange(_W2), indexing="ij")
    _IOTA_T[_r, :_H2, :_W2] = _off + _ii * _W2 + _jj
    _X0_T[_r, :_H2, :_W2] = _jj * _STRIDE
    _Y0_T[_r, :_H2, :_W2] = _ii * _STRIDE
    _X1_T[_r, :_H2, :_W2] = (_jj + _kw) * _STRIDE - 1
    _Y1_T[_r, :_H2, :_W2] = (_ii + _kh) * _STRIDE - 1
    _AR_T[_r, :_H2, :_W2] = float(_kw * _STRIDE) * float(_kh * _STRIDE)
    _off += _H2 * _W2
_GROUP_LO = [0, 1873, 3458]


def _body(x_ref, iota_ref, x0_ref, y0_ref, x1_ref, y1_ref, ar_ref,
          *refs):
    out_refs, idx_ref, sc_ref, acc_ref = refs[:13], refs[13], refs[14], refs[15]
    c = pl.program_id(1)

    @pl.when(c == 0)
    def _init():
        acc_ref[...] = jnp.zeros((_FM, _FM), jnp.float32)

    acc_ref[...] = acc_ref[...] + jnp.sum(x_ref[0], axis=0)

    @pl.when(c == _CB - 1)
    def _finish():
        s = acc_ref[...]
        hs = {1: s}
        cur = s
        for k in range(2, 11):
            cur = cur[:, : _FM + 1 - k] + s[:, k - 1:]
            hs[k] = cur
        maps = []
        for r, (kh, kw) in enumerate(_RATIOS):
            h = hs[kw]
            v = h
            for k in range(2, kh + 1):
                v = v[: _FM + 1 - k, :] + h[k - 1:, :]
            v = v / float(kh * kw)
            out_refs[r][0] = v
            maps.append(jnp.pad(v, ((0, kh - 1), (0, kw - 1)),
                                constant_values=-np.inf))

        lane = jax.lax.broadcasted_iota(jnp.int32, (1, 128), 1)
        idx_acc = jnp.zeros((1, 128), jnp.int32)
        sc_acc = jnp.zeros((1, 128), jnp.float32)
        neg = jnp.float32(-jnp.inf)
        col = 0
        for g, rats in enumerate(_GROUP_RATIOS):
            iot = {r: iota_ref[r] for r in rats}
            tx0 = {r: x0_ref[r] for r in rats}
            ty0 = {r: y0_ref[r] for r in rats}
            tx1 = {r: x1_ref[r] for r in rats}
            ty1 = {r: y1_ref[r] for r in rats}
            tar = {r: ar_ref[r] for r in rats}
            ms = {r: maps[r] for r in rats}
            last = jnp.int32(_GROUP_LO[g])
            for t in range(_CAT_NUMS[g]):
                m = jnp.max(jnp.stack([jnp.max(ms[r]) for r in rats]))
                valid = m != neg
                pick = jnp.min(jnp.stack(
                    [jnp.min(jnp.where(ms[r] == m, iot[r], _BIG_I))
                     for r in rats]))
                idx = jnp.where(valid, pick, last)
                eq = {r: iot[r] == idx for r in rats}
                bx0 = sum(jnp.sum(jnp.where(eq[r], tx0[r], 0.0)) for r in rats)
                by0 = sum(jnp.sum(jnp.where(eq[r], ty0[r], 0.0)) for r in rats)
                bx1 = sum(jnp.sum(jnp.where(eq[r], tx1[r], 0.0)) for r in rats)
                by1 = sum(jnp.sum(jnp.where(eq[r], ty1[r], 0.0)) for r in rats)
                bar = sum(jnp.sum(jnp.where(eq[r], tar[r], 0.0)) for r in rats)
                sel = sum(jnp.sum(jnp.where(eq[r], maps[r], 0.0)) for r in rats)
                idx_acc = jnp.where(lane == col + t, idx, idx_acc)
                sc_acc = jnp.where(lane == col + t, sel, sc_acc)
                for r in rats:
                    lx = (jnp.minimum(tx1[r], bx1)
                          - jnp.maximum(tx0[r], bx0) + 1.0)
                    ly = (jnp.minimum(ty1[r], by1)
                          - jnp.maximum(ty0[r], by0) + 1.0)
                    inter = jnp.where((lx < 0) | (ly < 0), 0.0, lx * ly)
                    iou = inter / (tar[r] + bar - inter)
                    kill = (iou > _IOU_THRESH) | eq[r]
                    ms[r] = jnp.where(jnp.logical_and(valid, kill), neg, ms[r])
                last = idx
            col += _CAT_NUMS[g]
        idx_ref[...] = idx_acc.reshape(1, 1, 128)
        sc_ref[...] = sc_acc.reshape(1, 1, 128)


def _run(x, proposalN):
    b = x.shape[0]
    tbl_spec = pl.BlockSpec((13, _FM, _FM), lambda i, c: (0, 0, 0))
    outs = pl.pallas_call(
        _body,
        grid=(b, _CB),
        in_specs=[pl.BlockSpec((1, _CBS, _FM, _FM), lambda i, c: (i, c, 0, 0)),
                  tbl_spec, tbl_spec, tbl_spec, tbl_spec, tbl_spec, tbl_spec],
        out_specs=[pl.BlockSpec((1, _FM + 1 - kh, _FM + 1 - kw),
                                lambda i, c: (i, 0, 0))
                   for (kh, kw) in _RATIOS]
                  + [pl.BlockSpec((1, 1, 128), lambda i, c: (i, 0, 0)),
                     pl.BlockSpec((1, 1, 128), lambda i, c: (i, 0, 0))],
        out_shape=[jax.ShapeDtypeStruct((b, _FM + 1 - kh, _FM + 1 - kw),
                                        jnp.float32)
                   for (kh, kw) in _RATIOS]
                  + [jax.ShapeDtypeStruct((b, 1, 128), jnp.int32),
                     jax.ShapeDtypeStruct((b, 1, 128), jnp.float32)],
        scratch_shapes=[pltpu.VMEM((_FM, _FM), jnp.float32)],
        compiler_params=pltpu.CompilerParams(
            dimension_semantics=("parallel", "arbitrary")),
    )(x, jnp.asarray(_IOTA_T), jnp.asarray(_X0_T), jnp.asarray(_Y0_T),
      jnp.asarray(_X1_T), jnp.asarray(_Y1_T), jnp.asarray(_AR_T))

    pooled, idx_o, sc_o = outs[:13], outs[13], outs[14]
    ws = jnp.concatenate([o.reshape(b, -1) for o in pooled], axis=1)
    inds = idx_o[:, 0, :7] + (jnp.asarray(proposalN, jnp.int32)
                              - sum(_CAT_NUMS))
    ssc = sc_o[:, 0, :7]
    return inds.astype(jnp.int32), ssc, ws


def kernel(x, proposalN):
    return _run(x, proposalN)


# speedup vs baseline: 1.1917x; 1.1917x over previous
import jax
import jax.numpy as jnp
import numpy as np
from jax.experimental import pallas as pl
from jax.experimental.pallas import tpu as pltpu

_RATIOS = [[4, 4], [3, 5], [5, 3], [6, 6], [5, 7], [7, 5], [8, 8], [6, 10],
           [10, 6], [7, 9], [9, 7], [7, 10], [10, 7]]
_STRIDE = 16
_FM = 28
_CAT_NUMS = [2, 3, 2]
_GROUP_RATIOS = [[0, 1, 2], [3, 4, 5], [6, 7, 8, 9, 10, 11, 12]]
_IOU_THRESH = 0.25
_CB = 6
_CBS = 128
_BIG_I = np.int32(1 << 30)

_IOTA_T = np.full((13, _FM, _FM), _BIG_I, np.int32)
_X0_T = np.full((13, _FM, _FM), 1e9, np.float32)
_Y0_T = np.full((13, _FM, _FM), 1e9, np.float32)
_X1_T = np.full((13, _FM, _FM), -1e9, np.float32)
_Y1_T = np.full((13, _FM, _FM), -1e9, np.float32)
_AR_T = np.full((13, _FM, _FM), 1.0, np.float32)
_OFFS = []
_off = 0
for _r, (_kh, _kw) in enumerate(_RATIOS):
    _H2, _W2 = _FM + 1 - _kh, _FM + 1 - _kw
    _OFFS.append(_off)
    _ii, _jj = np.meshgrid(np.arange(_H2), np.arange(_W2), indexing="ij")
    _IOTA_T[_r, :_H2, :_W2] = _off + _ii * _W2 + _jj
    _X0_T[_r, :_H2, :_W2] = _jj * _STRIDE
    _Y0_T[_r, :_H2, :_W2] = _ii * _STRIDE
    _X1_T[_r, :_H2, :_W2] = (_jj + _kw) * _STRIDE - 1
    _Y1_T[_r, :_H2, :_W2] = (_ii + _kh) * _STRIDE - 1
    _AR_T[_r, :_H2, :_W2] = float(_kw * _STRIDE) * float(_kh * _STRIDE)
    _off += _H2 * _W2
_GROUP_LO = [0, 1873, 3458]


def _body(x_ref, iota_ref, x0_ref, y0_ref, x1_ref, y1_ref, ar_ref,
          *refs):
    out_refs, idx_ref, sc_ref = refs[:13], refs[13], refs[14]

    def _finish():
        s = jnp.sum(x_ref[0], axis=0)
        hs = {1: s}
        cur = s
        for k in range(2, 11):
            cur = cur[:, : _FM + 1 - k] + s[:, k - 1:]
            hs[k] = cur
        maps = []
        for r, (kh, kw) in enumerate(_RATIOS):
            h = hs[kw]
            v = h
            for k in range(2, kh + 1):
                v = v[: _FM + 1 - k, :] + h[k - 1:, :]
            v = v / float(kh * kw)
            out_refs[r][0] = v
            maps.append(jnp.pad(v, ((0, kh - 1), (0, kw - 1)),
                                constant_values=-np.inf))

        lane = jax.lax.broadcasted_iota(jnp.int32, (1, 128), 1)
        idx_acc = jnp.zeros((1, 128), jnp.int32)
        sc_acc = jnp.zeros((1, 128), jnp.float32)
        neg = jnp.float32(-jnp.inf)
        col = 0
        for g, rats in enumerate(_GROUP_RATIOS):
            iot = {r: iota_ref[r] for r in rats}
            tx0 = {r: x0_ref[r] for r in rats}
            ty0 = {r: y0_ref[r] for r in rats}
            tx1 = {r: x1_ref[r] for r in rats}
            ty1 = {r: y1_ref[r] for r in rats}
            tar = {r: ar_ref[r] for r in rats}
            ms = {r: maps[r] for r in rats}
            last = jnp.int32(_GROUP_LO[g])
            for t in range(_CAT_NUMS[g]):
                m = jnp.max(jnp.stack([jnp.max(ms[r]) for r in rats]))
                valid = m != neg
                pick = jnp.min(jnp.stack(
                    [jnp.min(jnp.where(ms[r] == m, iot[r], _BIG_I))
                     for r in rats]))
                idx = jnp.where(valid, pick, last)
                eq = {r: iot[r] == idx for r in rats}
                bx0 = sum(jnp.sum(jnp.where(eq[r], tx0[r], 0.0)) for r in rats)
                by0 = sum(jnp.sum(jnp.where(eq[r], ty0[r], 0.0)) for r in rats)
                bx1 = sum(jnp.sum(jnp.where(eq[r], tx1[r], 0.0)) for r in rats)
                by1 = sum(jnp.sum(jnp.where(eq[r], ty1[r], 0.0)) for r in rats)
                bar = sum(jnp.sum(jnp.where(eq[r], tar[r], 0.0)) for r in rats)
                sel = sum(jnp.sum(jnp.where(eq[r], maps[r], 0.0)) for r in rats)
                idx_acc = jnp.where(lane == col + t, idx, idx_acc)
                sc_acc = jnp.where(lane == col + t, sel, sc_acc)
                for r in rats:
                    lx = (jnp.minimum(tx1[r], bx1)
                          - jnp.maximum(tx0[r], bx0) + 1.0)
                    ly = (jnp.minimum(ty1[r], by1)
                          - jnp.maximum(ty0[r], by0) + 1.0)
                    inter = jnp.where((lx < 0) | (ly < 0), 0.0, lx * ly)
                    iou = inter / (tar[r] + bar - inter)
                    kill = (iou > _IOU_THRESH) | eq[r]
                    ms[r] = jnp.where(jnp.logical_and(valid, kill), neg, ms[r])
                last = idx
            col += _CAT_NUMS[g]
        idx_ref[...] = idx_acc.reshape(1, 1, 128)
        sc_ref[...] = sc_acc.reshape(1, 1, 128)

    _finish()


def _run(x, proposalN):
    b = x.shape[0]
    tbl_spec = pl.BlockSpec((13, _FM, _FM), lambda i: (0, 0, 0))
    outs = pl.pallas_call(
        _body,
        grid=(b,),
        in_specs=[pl.BlockSpec((1, _CB * _CBS, _FM, _FM),
                               lambda i: (i, 0, 0, 0)),
                  tbl_spec, tbl_spec, tbl_spec, tbl_spec, tbl_spec, tbl_spec],
        out_specs=[pl.BlockSpec((1, _FM + 1 - kh, _FM + 1 - kw),
                                lambda i: (i, 0, 0))
                   for (kh, kw) in _RATIOS]
                  + [pl.BlockSpec((1, 1, 128), lambda i: (i, 0, 0)),
                     pl.BlockSpec((1, 1, 128), lambda i: (i, 0, 0))],
        out_shape=[jax.ShapeDtypeStruct((b, _FM + 1 - kh, _FM + 1 - kw),
                                        jnp.float32)
                   for (kh, kw) in _RATIOS]
                  + [jax.ShapeDtypeStruct((b, 1, 128), jnp.int32),
                     jax.ShapeDtypeStruct((b, 1, 128), jnp.float32)],
        compiler_params=pltpu.CompilerParams(
            dimension_semantics=("parallel",)),
    )(x, jnp.asarray(_IOTA_T), jnp.asarray(_X0_T), jnp.asarray(_Y0_T),
      jnp.asarray(_X1_T), jnp.asarray(_Y1_T), jnp.asarray(_AR_T))

    pooled, idx_o, sc_o = outs[:13], outs[13], outs[14]
    ws = jnp.concatenate([o.reshape(b, -1) for o in pooled], axis=1)
    inds = idx_o[:, 0, :7] + (jnp.asarray(proposalN, jnp.int32)
                              - sum(_CAT_NUMS))
    ssc = sc_o[:, 0, :7]
    return inds.astype(jnp.int32), ssc, ws


def kernel(x, proposalN):
    return _run(x, proposalN)


# speedup vs baseline: 1.2373x; 1.0382x over previous
import jax
import jax.numpy as jnp
import numpy as np
from jax.experimental import pallas as pl
from jax.experimental.pallas import tpu as pltpu

_RATIOS = [[4, 4], [3, 5], [5, 3], [6, 6], [5, 7], [7, 5], [8, 8], [6, 10],
           [10, 6], [7, 9], [9, 7], [7, 10], [10, 7]]
_STRIDE = 16
_FM = 28
_CAT_NUMS = [2, 3, 2]
_GROUP_RATIOS = [[0, 1, 2], [3, 4, 5], [6, 7, 8, 9, 10, 11, 12]]
_GROUP_LO = [0, 1873, 3458]
_IOU_THRESH = 0.25
_NCHUNK = 3
_CHW = 256
_BIG_I = np.int32(1 << 30)

_IOTA_T = np.full((13, _FM, _FM), _BIG_I, np.int32)
_off = 0
for _r, (_kh, _kw) in enumerate(_RATIOS):
    _H2, _W2 = _FM + 1 - _kh, _FM + 1 - _kw
    _ii, _jj = np.meshgrid(np.arange(_H2), np.arange(_W2), indexing="ij")
    _IOTA_T[_r, :_H2, :_W2] = _off + _ii * _W2 + _jj
    _off += _H2 * _W2

_X0_S = np.zeros((_off,), np.float32)
_Y0_S = np.zeros((_off,), np.float32)
_X1_S = np.zeros((_off,), np.float32)
_Y1_S = np.zeros((_off,), np.float32)
_AR_S = np.zeros((_off,), np.float32)
_o = 0
for (_kh, _kw) in _RATIOS:
    _H2, _W2 = _FM + 1 - _kh, _FM + 1 - _kw
    _ii, _jj = np.meshgrid(np.arange(_H2), np.arange(_W2), indexing="ij")
    _n = _H2 * _W2
    _X0_S[_o:_o + _n] = (_jj * _STRIDE).ravel()
    _Y0_S[_o:_o + _n] = (_ii * _STRIDE).ravel()
    _X1_S[_o:_o + _n] = ((_jj + _kw) * _STRIDE - 1).ravel()
    _Y1_S[_o:_o + _n] = ((_ii + _kh) * _STRIDE - 1).ravel()
    _AR_S[_o:_o + _n] = float(_kw * _STRIDE) * float(_kh * _STRIDE)
    _o += _n

_X0_T = np.full((13, _FM, _FM), 1e9, np.float32)
_Y0_T = np.full((13, _FM, _FM), 1e9, np.float32)
_X1_T = np.full((13, _FM, _FM), -1e9, np.float32)
_Y1_T = np.full((13, _FM, _FM), -1e9, np.float32)
_AR_T = np.full((13, _FM, _FM), 1.0, np.float32)
for _r, (_kh, _kw) in enumerate(_RATIOS):
    _H2, _W2 = _FM + 1 - _kh, _FM + 1 - _kw
    _ii, _jj = np.meshgrid(np.arange(_H2), np.arange(_W2), indexing="ij")
    _X0_T[_r, :_H2, :_W2] = _jj * _STRIDE
    _Y0_T[_r, :_H2, :_W2] = _ii * _STRIDE
    _X1_T[_r, :_H2, :_W2] = (_jj + _kw) * _STRIDE - 1
    _Y1_T[_r, :_H2, :_W2] = (_ii + _kh) * _STRIDE - 1
    _AR_T[_r, :_H2, :_W2] = float(_kw * _STRIDE) * float(_kh * _STRIDE)


def _body(x_ref, iota_ref, x0v_ref, y0v_ref, x1v_ref, y1v_ref, arv_ref,
          x0s_ref, y0s_ref, x1s_ref, y1s_ref, ars_ref, *refs):
    out_refs, idx_ref, sc_ref, acc_ref = refs[:13], refs[13], refs[14], refs[15]
    c = pl.program_id(1)

    @pl.when(c == 0)
    def _init():
        acc_ref[...] = jnp.zeros((_FM, _FM), jnp.float32)

    acc_ref[...] = acc_ref[...] + jnp.sum(x_ref[0], axis=0)

    @pl.when(c == _NCHUNK - 1)
    def _finish():
        s = acc_ref[...]
        hs = {1: s}
        cur = s
        for k in range(2, 11):
            cur = cur[:, : _FM + 1 - k] + s[:, k - 1:]
            hs[k] = cur
        maps = []
        for r, (kh, kw) in enumerate(_RATIOS):
            h = hs[kw]
            v = h
            for k in range(2, kh + 1):
                v = v[: _FM + 1 - k, :] + h[k - 1:, :]
            v = v / float(kh * kw)
            out_refs[r][0] = v
            maps.append(jnp.pad(v, ((0, kh - 1), (0, kw - 1)),
                                constant_values=-np.inf))

        lane = jax.lax.broadcasted_iota(jnp.int32, (1, 128), 1)
        idx_acc = jnp.zeros((1, 128), jnp.int32)
        sc_acc = jnp.zeros((1, 128), jnp.float32)
        neg = jnp.float32(-jnp.inf)
        col = 0
        for g, rats in enumerate(_GROUP_RATIOS):
            iot = {r: iota_ref[r] for r in rats}
            ms = {r: maps[r] for r in rats}
            last = jnp.int32(_GROUP_LO[g])
            prev_sel = neg
            for t in range(_CAT_NUMS[g]):
                mm = ms[rats[0]]
                for r in rats[1:]:
                    mm = jnp.maximum(mm, ms[r])
                m = jnp.max(mm)
                valid = m != neg
                cand = jnp.where(ms[rats[0]] == m, iot[rats[0]], _BIG_I)
                for r in rats[1:]:
                    cand = jnp.minimum(cand,
                                       jnp.where(ms[r] == m, iot[r], _BIG_I))
                pick = jnp.min(cand)
                idx = jnp.where(valid, pick, last)
                sel = jnp.where(valid, m, prev_sel)
                idx_acc = jnp.where(lane == col + t, idx, idx_acc)
                sc_acc = jnp.where(lane == col + t, sel, sc_acc)
                if t < _CAT_NUMS[g] - 1:
                    bx0 = x0s_ref[idx]
                    by0 = y0s_ref[idx]
                    bx1 = x1s_ref[idx]
                    by1 = y1s_ref[idx]
                    bar = ars_ref[idx]
                    for r in rats:
                        lx = (jnp.minimum(x1v_ref[r], bx1)
                              - jnp.maximum(x0v_ref[r], bx0) + 1.0)
                        ly = (jnp.minimum(y1v_ref[r], by1)
                              - jnp.maximum(y0v_ref[r], by0) + 1.0)
                        inter = jnp.where((lx < 0) | (ly < 0), 0.0, lx * ly)
                        iou = inter / (arv_ref[r] + bar - inter)
                        kill = (iou > _IOU_THRESH) | (iot[r] == idx)
                        ms[r] = jnp.where(jnp.logical_and(valid, kill),
                                          neg, ms[r])
                last = idx
                prev_sel = sel
            col += _CAT_NUMS[g]
        idx_ref[...] = idx_acc.reshape(1, 1, 128)
        sc_ref[...] = sc_acc.reshape(1, 1, 128)


def _run(x, proposalN):
    b = x.shape[0]
    tbl_spec = pl.BlockSpec((13, _FM, _FM), lambda i, c: (0, 0, 0))
    smem_spec = pl.BlockSpec(memory_space=pltpu.SMEM)
    outs = pl.pallas_call(
        _body,
        grid=(b, _NCHUNK),
        in_specs=[pl.BlockSpec((1, _CHW, _FM, _FM),
                               lambda i, c: (i, c, 0, 0)),
                  tbl_spec, tbl_spec, tbl_spec, tbl_spec, tbl_spec, tbl_spec,
                  smem_spec, smem_spec, smem_spec, smem_spec, smem_spec],
        out_specs=[pl.BlockSpec((1, _FM + 1 - kh, _FM + 1 - kw),
                                lambda i, c: (i, 0, 0))
                   for (kh, kw) in _RATIOS]
                  + [pl.BlockSpec((1, 1, 128), lambda i, c: (i, 0, 0)),
                     pl.BlockSpec((1, 1, 128), lambda i, c: (i, 0, 0))],
        out_shape=[jax.ShapeDtypeStruct((b, _FM + 1 - kh, _FM + 1 - kw),
                                        jnp.float32)
                   for (kh, kw) in _RATIOS]
                  + [jax.ShapeDtypeStruct((b, 1, 128), jnp.int32),
                     jax.ShapeDtypeStruct((b, 1, 128), jnp.float32)],
        scratch_shapes=[pltpu.VMEM((_FM, _FM), jnp.float32)],
        compiler_params=pltpu.CompilerParams(
            dimension_semantics=("parallel", "arbitrary")),
    )(x, jnp.asarray(_IOTA_T), jnp.asarray(_X0_T), jnp.asarray(_Y0_T),
      jnp.asarray(_X1_T), jnp.asarray(_Y1_T), jnp.asarray(_AR_T),
      jnp.asarray(_X0_S), jnp.asarray(_Y0_S), jnp.asarray(_X1_S),
      jnp.asarray(_Y1_S), jnp.asarray(_AR_S))

    pooled, idx_o, sc_o = outs[:13], outs[13], outs[14]
    ws = jnp.concatenate([o.reshape(b, -1) for o in pooled], axis=1)
    inds = idx_o[:, 0, :7] + (jnp.asarray(proposalN, jnp.int32)
                              - sum(_CAT_NUMS))
    ssc = sc_o[:, 0, :7]
    return inds.astype(jnp.int32), ssc, ws


def kernel(x, proposalN):
    return _run(x, proposalN)


# speedup vs baseline: 2.1113x; 1.7064x over previous
import jax
import jax.numpy as jnp
import numpy as np
from jax.experimental import pallas as pl
from jax.experimental.pallas import tpu as pltpu

_RATIOS = [[4, 4], [3, 5], [5, 3], [6, 6], [5, 7], [7, 5], [8, 8], [6, 10],
           [10, 6], [7, 9], [9, 7], [7, 10], [10, 7]]
_STRIDE = 16
_FM = 28
_CAT_NUMS = [2, 3, 2]
_GROUP_RATIOS = [[0, 1, 2], [3, 4, 5], [6, 7, 8, 9, 10, 11, 12]]
_GROUP_LO = [0, 1873, 3458]
_IOU_THRESH = 0.25
_NCHUNK = 3
_CHW = 256
_BIG_I = np.int32(1 << 30)

_IOTA_T = np.full((13, _FM, _FM), _BIG_I, np.int32)
_off = 0
for _r, (_kh, _kw) in enumerate(_RATIOS):
    _H2, _W2 = _FM + 1 - _kh, _FM + 1 - _kw
    _ii, _jj = np.meshgrid(np.arange(_H2), np.arange(_W2), indexing="ij")
    _IOTA_T[_r, :_H2, :_W2] = _off + _ii * _W2 + _jj
    _off += _H2 * _W2

_X0_S = np.zeros((_off,), np.float32)
_Y0_S = np.zeros((_off,), np.float32)
_X1_S = np.zeros((_off,), np.float32)
_Y1_S = np.zeros((_off,), np.float32)
_AR_S = np.zeros((_off,), np.float32)
_o = 0
for (_kh, _kw) in _RATIOS:
    _H2, _W2 = _FM + 1 - _kh, _FM + 1 - _kw
    _ii, _jj = np.meshgrid(np.arange(_H2), np.arange(_W2), indexing="ij")
    _n = _H2 * _W2
    _X0_S[_o:_o + _n] = (_jj * _STRIDE).ravel()
    _Y0_S[_o:_o + _n] = (_ii * _STRIDE).ravel()
    _X1_S[_o:_o + _n] = ((_jj + _kw) * _STRIDE - 1).ravel()
    _Y1_S[_o:_o + _n] = ((_ii + _kh) * _STRIDE - 1).ravel()
    _AR_S[_o:_o + _n] = float(_kw * _STRIDE) * float(_kh * _STRIDE)
    _o += _n

_X0_T = np.full((13, _FM, _FM), 1e9, np.float32)
_Y0_T = np.full((13, _FM, _FM), 1e9, np.float32)
_X1_T = np.full((13, _FM, _FM), -1e9, np.float32)
_Y1_T = np.full((13, _FM, _FM), -1e9, np.float32)
_AR_T = np.full((13, _FM, _FM), 1.0, np.float32)
for _r, (_kh, _kw) in enumerate(_RATIOS):
    _H2, _W2 = _FM + 1 - _kh, _FM + 1 - _kw
    _ii, _jj = np.meshgrid(np.arange(_H2), np.arange(_W2), indexing="ij")
    _X0_T[_r, :_H2, :_W2] = _jj * _STRIDE
    _Y0_T[_r, :_H2, :_W2] = _ii * _STRIDE
    _X1_T[_r, :_H2, :_W2] = (_jj + _kw) * _STRIDE - 1
    _Y1_T[_r, :_H2, :_W2] = (_ii + _kh) * _STRIDE - 1
    _AR_T[_r, :_H2, :_W2] = float(_kw * _STRIDE) * float(_kh * _STRIDE)


def _body(x_ref, iota_ref, x0v_ref, y0v_ref, x1v_ref, y1v_ref, arv_ref,
          x0s_ref, y0s_ref, x1s_ref, y1s_ref, ars_ref, *refs):
    out_refs, idx_ref, sc_ref = refs[:13], refs[13], refs[14]

    def _finish():
        s784 = jnp.sum(x_ref[0], axis=0)
        s = jnp.concatenate(
            [s784[None, _FM * i: _FM * (i + 1)] for i in range(_FM)], axis=0)
        hs = {1: s}
        cur = s
        for k in range(2, 11):
            cur = cur[:, : _FM + 1 - k] + s[:, k - 1:]
            hs[k] = cur
        maps = []
        for r, (kh, kw) in enumerate(_RATIOS):
            h = hs[kw]
            v = h
            for k in range(2, kh + 1):
                v = v[: _FM + 1 - k, :] + h[k - 1:, :]
            v = v / float(kh * kw)
            out_refs[r][0] = v
            maps.append(jnp.pad(v, ((0, kh - 1), (0, kw - 1)),
                                constant_values=-np.inf))

        lane = jax.lax.broadcasted_iota(jnp.int32, (1, 128), 1)
        idx_acc = jnp.zeros((1, 128), jnp.int32)
        sc_acc = jnp.zeros((1, 128), jnp.float32)
        neg = jnp.float32(-jnp.inf)
        col = 0
        for g, rats in enumerate(_GROUP_RATIOS):
            iot = {r: iota_ref[r] for r in rats}
            ms = {r: maps[r] for r in rats}
            last = jnp.int32(_GROUP_LO[g])
            prev_sel = neg
            for t in range(_CAT_NUMS[g]):
                mm = ms[rats[0]]
                for r in rats[1:]:
                    mm = jnp.maximum(mm, ms[r])
                m = jnp.max(mm)
                valid = m != neg
                cand = jnp.where(ms[rats[0]] == m, iot[rats[0]], _BIG_I)
                for r in rats[1:]:
                    cand = jnp.minimum(cand,
                                       jnp.where(ms[r] == m, iot[r], _BIG_I))
                pick = jnp.min(cand)
                idx = jnp.where(valid, pick, last)
                sel = jnp.where(valid, m, prev_sel)
                idx_acc = jnp.where(lane == col + t, idx, idx_acc)
                sc_acc = jnp.where(lane == col + t, sel, sc_acc)
                if t < _CAT_NUMS[g] - 1:
                    bx0 = x0s_ref[idx]
                    by0 = y0s_ref[idx]
                    bx1 = x1s_ref[idx]
                    by1 = y1s_ref[idx]
                    bar = ars_ref[idx]
                    for r in rats:
                        lx = (jnp.minimum(x1v_ref[r], bx1)
                              - jnp.maximum(x0v_ref[r], bx0) + 1.0)
                        ly = (jnp.minimum(y1v_ref[r], by1)
                              - jnp.maximum(y0v_ref[r], by0) + 1.0)
                        inter = jnp.where((lx < 0) | (ly < 0), 0.0, lx * ly)
                        iou = inter / (arv_ref[r] + bar - inter)
                        kill = (iou > _IOU_THRESH) | (iot[r] == idx)
                        ms[r] = jnp.where(jnp.logical_and(valid, kill),
                                          neg, ms[r])
                last = idx
                prev_sel = sel
            col += _CAT_NUMS[g]
        idx_ref[...] = idx_acc.reshape(1, 1, 128)
        sc_ref[...] = sc_acc.reshape(1, 1, 128)

    _finish()


def _run(x, proposalN):
    b = x.shape[0]
    ch = x.shape[1]
    y = x.reshape(b, ch, _FM * _FM)
    tbl_spec = pl.BlockSpec((13, _FM, _FM), lambda i: (0, 0, 0))
    smem_spec = pl.BlockSpec(memory_space=pltpu.SMEM)
    outs = pl.pallas_call(
        _body,
        grid=(b,),
        in_specs=[pl.BlockSpec((1, ch, _FM * _FM), lambda i: (i, 0, 0)),
                  tbl_spec, tbl_spec, tbl_spec, tbl_spec, tbl_spec, tbl_spec,
                  smem_spec, smem_spec, smem_spec, smem_spec, smem_spec],
        out_specs=[pl.BlockSpec((1, _FM + 1 - kh, _FM + 1 - kw),
                                lambda i: (i, 0, 0))
                   for (kh, kw) in _RATIOS]
                  + [pl.BlockSpec((1, 1, 128), lambda i: (i, 0, 0)),
                     pl.BlockSpec((1, 1, 128), lambda i: (i, 0, 0))],
        out_shape=[jax.ShapeDtypeStruct((b, _FM + 1 - kh, _FM + 1 - kw),
                                        jnp.float32)
                   for (kh, kw) in _RATIOS]
                  + [jax.ShapeDtypeStruct((b, 1, 128), jnp.int32),
                     jax.ShapeDtypeStruct((b, 1, 128), jnp.float32)],
        compiler_params=pltpu.CompilerParams(
            dimension_semantics=("parallel",)),
    )(y, jnp.asarray(_IOTA_T), jnp.asarray(_X0_T), jnp.asarray(_Y0_T),
      jnp.asarray(_X1_T), jnp.asarray(_Y1_T), jnp.asarray(_AR_T),
      jnp.asarray(_X0_S), jnp.asarray(_Y0_S), jnp.asarray(_X1_S),
      jnp.asarray(_Y1_S), jnp.asarray(_AR_S))

    pooled, idx_o, sc_o = outs[:13], outs[13], outs[14]
    ws = jnp.concatenate([o.reshape(b, -1) for o in pooled], axis=1)
    inds = idx_o[:, 0, :7] + (jnp.asarray(proposalN, jnp.int32)
                              - sum(_CAT_NUMS))
    ssc = sc_o[:, 0, :7]
    return inds.astype(jnp.int32), ssc, ws


def kernel(x, proposalN):
    return _run(x, proposalN)
